# baseline (speedup 1.0000x reference)
import jax
import jax.numpy as jnp
from jax.experimental import pallas as pl

N_TOKENS = 16384
EMBED_DIM = 2048
LOW_RANK_DIM = 16
OUT_DIM = 64

BLOCK_T = 512


def _fused_body(x_ref, w1t_ref, w2t_ref, out_ref):
    h = jnp.dot(x_ref[...], w1t_ref[...], preferred_element_type=jnp.float32)
    out_ref[...] = jnp.dot(h, w2t_ref[...], preferred_element_type=jnp.float32)


def kernel(x, w1, w2):
    n = x.shape[0]
    w1t = w1.T
    w2t = w2.T
    grid = (n // BLOCK_T,)
    return pl.pallas_call(
        _fused_body,
        grid=grid,
        in_specs=[
            pl.BlockSpec((BLOCK_T, EMBED_DIM), lambda i: (i, 0)),
            pl.BlockSpec((EMBED_DIM, LOW_RANK_DIM), lambda i: (0, 0)),
            pl.BlockSpec((LOW_RANK_DIM, OUT_DIM), lambda i: (0, 0)),
        ],
        out_specs=pl.BlockSpec((BLOCK_T, OUT_DIM), lambda i: (i, 0)),
        out_shape=jax.ShapeDtypeStruct((n, OUT_DIM), jnp.float32),
    )(x, w1t, w2t)


# speedup vs baseline: 1.1640x; 1.1640x over previous
import jax
import jax.numpy as jnp
from jax.experimental import pallas as pl
from jax.experimental.pallas import tpu as pltpu

N_TOKENS = 16384
EMBED_DIM = 2048
LOW_RANK_DIM = 16
OUT_DIM = 64

BLOCK_T = 2048


def _fused_body(x_ref, w1t_ref, w2t_ref, out_ref):
    h = jnp.dot(x_ref[...], w1t_ref[...], preferred_element_type=jnp.float32)
    out_ref[...] = jnp.dot(h, w2t_ref[...], preferred_element_type=jnp.float32)


def kernel(x, w1, w2):
    n = x.shape[0]
    w1t = w1.T
    w2t = w2.T
    grid = (n // BLOCK_T,)
    return pl.pallas_call(
        _fused_body,
        grid=grid,
        in_specs=[
            pl.BlockSpec((BLOCK_T, EMBED_DIM), lambda i: (i, 0)),
            pl.BlockSpec((EMBED_DIM, LOW_RANK_DIM), lambda i: (0, 0)),
            pl.BlockSpec((LOW_RANK_DIM, OUT_DIM), lambda i: (0, 0)),
        ],
        out_specs=pl.BlockSpec((BLOCK_T, OUT_DIM), lambda i: (i, 0)),
        out_shape=jax.ShapeDtypeStruct((n, OUT_DIM), jnp.float32),
        compiler_params=pltpu.CompilerParams(
            dimension_semantics=("parallel",),
        ),
    )(x, w1t, w2t)
